# baseline (speedup 1.0000x reference)
import functools

import jax
import jax.numpy as jnp
from jax import lax
from jax.experimental import pallas as pl
from jax.experimental.pallas import tpu as pltpu
from jax.experimental.pallas import tpu_sc as plsc

D_VOCAB = 1_000_000
D_MODEL = 64
N_TOK = 4096 * 50

NC, NS = 2, 16
NW = NC * NS
TOK_PER_W = N_TOK // NW
CHUNK = 128
NCHUNK = TOK_PER_W // CHUNK

K = 256
NG = D_VOCAB // K
TAIL_START = D_VOCAB - K
NCH = 124

_MESH = plsc.VectorSubcoreMesh(core_axis_name="c", subcore_axis_name="s")
_NOTILE = pltpu.CompilerParams(use_tc_tiling_on_sc=False)
_NOLAYOUT = pltpu.CompilerParams(
    use_tc_tiling_on_sc=False, needs_layout_passes=False
)


@functools.partial(
    pl.kernel,
    out_type=jax.ShapeDtypeStruct((D_VOCAB, D_MODEL), jnp.float32),
    mesh=_MESH,
    scratch_types=[
        pltpu.VMEM((D_MODEL, K), jnp.float32),
        pltpu.VMEM((D_MODEL, K), jnp.float32),
        pltpu.VMEM((K, D_MODEL), jnp.float32),
        pltpu.VMEM((K, D_MODEL), jnp.float32),
        pltpu.SemaphoreType.DMA,
        pltpu.SemaphoreType.DMA,
        pltpu.SemaphoreType.DMA,
        pltpu.SemaphoreType.DMA,
    ],
    compiler_params=_NOLAYOUT,
)
def _sc_transpose(w_hbm, wt_hbm, in0, in1, ot0, ot1, is0, is1, os0, os1):
    w = lax.axis_index("s") * NC + lax.axis_index("c")

    def chunk_start(c):
        g = w + NW * c
        tail = jnp.logical_and(w == NW - 1, c == NCH - 1)
        return jnp.where(g < NG, K * g, jnp.where(tail, TAIL_START, 0))

    @pl.loop(0, NCH)
    def _(c):
        start = chunk_start(c)
        pltpu.sync_copy(w_hbm.at[:, pl.ds(start, K)], in0)

        @pl.loop(0, D_MODEL)
        def _(d):
            col = jnp.full((16,), d, dtype=jnp.int32)
            for g in range(K // 16):
                vals = in0[d, pl.ds(g * 16, 16)]
                row = jnp.arange(16, dtype=jnp.int32) + g * 16
                plsc.store_scatter(ot0, [row, col], vals)

        pltpu.sync_copy(ot0, wt_hbm.at[pl.ds(start, K)])


@functools.partial(
    pl.kernel,
    out_type=jax.ShapeDtypeStruct((N_TOK, D_MODEL), jnp.float32),
    mesh=_MESH,
    scratch_types=[
        pltpu.VMEM((NCHUNK, CHUNK), jnp.int32),
        pltpu.VMEM((CHUNK, D_MODEL), jnp.float32),
        pltpu.VMEM((CHUNK, D_MODEL), jnp.float32),
        pltpu.SemaphoreType.DMA,
        pltpu.SemaphoreType.DMA,
        pltpu.SemaphoreType.DMA,
        pltpu.SemaphoreType.DMA,
    ],
    compiler_params=_NOTILE,
)
def _sc_gather(x_hbm, wt_hbm, out_hbm, idx_v, r0, r1, gs0, gs1, ss0, ss1):
    w = lax.axis_index("s") * NC + lax.axis_index("c")
    pltpu.sync_copy(x_hbm.at[w], idx_v)

    def body(c, carry):
        pltpu.async_copy(wt_hbm.at[idx_v.at[c]], r0, gs0).wait()
        pltpu.sync_copy(r0, out_hbm.at[pl.ds(w * TOK_PER_W + c * CHUNK, CHUNK)])
        return carry

    lax.fori_loop(0, NCHUNK, body, 0)


def kernel(x, W_E):
    w_t = _sc_transpose(W_E)
    x3 = x.reshape(NW, NCHUNK, CHUNK).astype(jnp.int32)
    out = _sc_gather(x3, w_t)
    return out.reshape(4096, 50, D_MODEL)


# speedup vs baseline: 1.0386x; 1.0386x over previous
import functools

import jax
import jax.numpy as jnp
from jax import lax
from jax.experimental import pallas as pl
from jax.experimental.pallas import tpu as pltpu
from jax.experimental.pallas import tpu_sc as plsc

D_VOCAB = 1_000_000
D_MODEL = 64
N_TOK = 4096 * 50

NC, NS = 2, 16
NW = NC * NS
TOK_PER_W = N_TOK // NW
CHUNK = 128
NCHUNK = TOK_PER_W // CHUNK

K = 256
NG = D_VOCAB // K
TAIL_START = D_VOCAB - K
NCH = 124

_MESH = plsc.VectorSubcoreMesh(core_axis_name="c", subcore_axis_name="s")
_NOTILE = pltpu.CompilerParams(use_tc_tiling_on_sc=False)
_NOLAYOUT = pltpu.CompilerParams(
    use_tc_tiling_on_sc=False, needs_layout_passes=False
)


@functools.partial(
    pl.kernel,
    out_type=jax.ShapeDtypeStruct((D_VOCAB, D_MODEL), jnp.float32),
    mesh=_MESH,
    scratch_types=[
        pltpu.VMEM((D_MODEL, K), jnp.float32),
        pltpu.VMEM((D_MODEL, K), jnp.float32),
        pltpu.VMEM((K, D_MODEL), jnp.float32),
        pltpu.VMEM((K, D_MODEL), jnp.float32),
        pltpu.SemaphoreType.DMA,
        pltpu.SemaphoreType.DMA,
        pltpu.SemaphoreType.DMA,
        pltpu.SemaphoreType.DMA,
    ],
    compiler_params=_NOLAYOUT,
)
def _sc_transpose(w_hbm, wt_hbm, in0, in1, ot0, ot1, is0, is1, os0, os1):
    w = lax.axis_index("s") * NC + lax.axis_index("c")

    def chunk_start(c):
        g = w + NW * c
        tail = jnp.logical_and(w == NW - 1, c == NCH - 1)
        return jnp.where(g < NG, K * g, jnp.where(tail, TAIL_START, 0))

    ins, ots, iss, oss = [in0, in1], [ot0, ot1], [is0, is1], [os0, os1]

    pltpu.async_copy(w_hbm.at[:, pl.ds(chunk_start(0), K)], in0, is0)

    @pl.loop(0, NCH, step=2)
    def _(cc):
        for b in range(2):
            c = cc + b

            @pl.when(c + 1 < NCH)
            def _():
                pltpu.async_copy(
                    w_hbm.at[:, pl.ds(chunk_start(c + 1), K)], ins[1 - b], iss[1 - b]
                )

            pltpu.make_async_copy(
                w_hbm.at[:, pl.ds(0, K)], ins[b], iss[b]
            ).wait()

            @pl.when(c >= 2)
            def _():
                pltpu.make_async_copy(
                    ots[b], wt_hbm.at[pl.ds(0, K)], oss[b]
                ).wait()

            @pl.loop(0, D_MODEL)
            def _(d):
                col = jnp.full((16,), d, dtype=jnp.int32)
                vals = [ins[b][d, pl.ds(g * 16, 16)] for g in range(K // 16)]
                for g in range(K // 16):
                    row = jnp.arange(16, dtype=jnp.int32) + g * 16
                    plsc.store_scatter(ots[b], [row, col], vals[g])

            pltpu.async_copy(ots[b], wt_hbm.at[pl.ds(chunk_start(c), K)], oss[b])

    for b in range(2):
        pltpu.make_async_copy(ots[b], wt_hbm.at[pl.ds(0, K)], oss[b]).wait()


@functools.partial(
    pl.kernel,
    out_type=jax.ShapeDtypeStruct((N_TOK, D_MODEL), jnp.float32),
    mesh=_MESH,
    scratch_types=[
        pltpu.VMEM((NCHUNK, CHUNK), jnp.int32),
        pltpu.VMEM((CHUNK, D_MODEL), jnp.float32),
        pltpu.VMEM((CHUNK, D_MODEL), jnp.float32),
        pltpu.SemaphoreType.DMA,
        pltpu.SemaphoreType.DMA,
        pltpu.SemaphoreType.DMA,
        pltpu.SemaphoreType.DMA,
    ],
    compiler_params=_NOTILE,
)
def _sc_gather(x_hbm, wt_hbm, out_hbm, idx_v, r0, r1, gs0, gs1, ss0, ss1):
    w = lax.axis_index("s") * NC + lax.axis_index("c")
    pltpu.sync_copy(x_hbm.at[w], idx_v)

    def body(c, carry):
        pltpu.async_copy(wt_hbm.at[idx_v.at[c]], r0, gs0).wait()
        pltpu.sync_copy(r0, out_hbm.at[pl.ds(w * TOK_PER_W + c * CHUNK, CHUNK)])
        return carry

    lax.fori_loop(0, NCHUNK, body, 0)


def kernel(x, W_E):
    w_t = _sc_transpose(W_E)
    x3 = x.reshape(NW, NCHUNK, CHUNK).astype(jnp.int32)
    out = _sc_gather(x3, w_t)
    return out.reshape(4096, 50, D_MODEL)
